# baseline (speedup 1.0000x reference)
import jax
import jax.numpy as jnp
from jax import lax
from jax.experimental import pallas as pl
from jax.experimental.pallas import tpu as pltpu
from jax.experimental.pallas import tpu_sc as plsc

T = 50
B = 1024
EMB = 100
EMBP = 128
HID = 128
OUT = 2
BC = 1024
GW = 128


def _sc_gather_pairs(emb2, idx2):
    ni = idx2.shape[1]
    d2 = emb2.shape[1]
    mesh = plsc.VectorSubcoreMesh(core_axis_name="core",
                                  subcore_axis_name="subcore")

    @pl.kernel(out_type=jax.ShapeDtypeStruct((ni, d2), jnp.float32),
               mesh=mesh,
               compiler_params=pltpu.CompilerParams(
                   use_tc_tiling_on_sc=False))
    def k(emb_hbm, idx_hbm, out_hbm):
        def body(i_vmem, o_vmem):
            pltpu.sync_copy(emb_hbm.at[i_vmem.at[0]], o_vmem)

        pltpu.emit_pipeline(
            body,
            grid=(ni // GW,),
            in_specs=[pl.BlockSpec((1, GW), index_map=lambda i: (0, i))],
            out_specs=[pl.BlockSpec((GW, d2),
                                    index_map=lambda i: (i, 0))],
            core_axis_name=("core", "subcore"),
            dimension_semantics=(pltpu.PARALLEL,),
        )(idx_hbm, out_hbm)

    return k(emb2, idx2)


def _sel_cast(pairs, par):
    def body(x_ref, p_ref, o_ref):
        x = x_ref[0]
        sel = jnp.where(p_ref[0] > 0, x[:, EMB:2 * EMB], x[:, 0:EMB])
        o_ref[0, :, 0:EMB] = sel.astype(jnp.bfloat16)
        o_ref[0, :, EMB:EMBP] = jnp.zeros((B, EMBP - EMB), jnp.bfloat16)

    return pl.pallas_call(
        body,
        grid=(T,),
        in_specs=[pl.BlockSpec((1, B, 2 * EMB), lambda i: (i, 0, 0)),
                  pl.BlockSpec((1, B, 1), lambda i: (i, 0, 0))],
        out_specs=pl.BlockSpec((1, B, EMBP), lambda i: (i, 0, 0)),
        out_shape=jax.ShapeDtypeStruct((T, B, EMBP), jnp.bfloat16),
        compiler_params=pltpu.CompilerParams(
            dimension_semantics=("arbitrary",)),
    )(pairs, par)


def _lstm_body(x_ref, w0f_ref, b0f_ref, w0r_ref, b0r_ref,
               wih1f_ref, whh1f_ref, b1f_ref, wih1r_ref, whh1r_ref, b1r_ref,
               fcwf_ref, fcwr_ref, fcb_ref, out_ref,
               ys0_ref, xhf_ref, xhr_ref, cf_ref, cr_ref, h1f_ref, h1r_ref):
    f32 = jnp.float32
    bf16 = jnp.bfloat16

    def update(g, c):
        ti = jnp.tanh(g[:, 0:HID])
        tf = jnp.tanh(g[:, HID:2 * HID])
        tg = jnp.tanh(g[:, 2 * HID:3 * HID])
        to = jnp.tanh(g[:, 3 * HID:4 * HID])
        c = 0.5 * ((tf * c + c) + (ti * tg + tg))
        tc = jnp.tanh(c)
        h = 0.5 * (to * tc + tc)
        return h, c

    zb = jnp.zeros((BC, HID), bf16)
    zf = jnp.zeros((BC, HID), f32)
    xhf_ref[:, HID:2 * HID] = zb
    xhr_ref[:, HID:2 * HID] = zb
    cf_ref[...] = zf
    cr_ref[...] = zf

    def step0(i):
        tr = T - 1 - i
        xhf_ref[:, 0:HID] = x_ref[i].astype(bf16)
        xhr_ref[:, 0:HID] = x_ref[tr].astype(bf16)
        gf = (jnp.dot(xhf_ref[...], w0f_ref[...], preferred_element_type=f32)
              + b0f_ref[...])
        gr = (jnp.dot(xhr_ref[...], w0r_ref[...], preferred_element_type=f32)
              + b0r_ref[...])
        hf, cf = update(gf, cf_ref[...])
        hr, cr = update(gr, cr_ref[...])
        hfb = hf.astype(bf16)
        hrb = hr.astype(bf16)
        ys0_ref[i, :, 0:HID] = hfb
        ys0_ref[tr, :, HID:2 * HID] = hrb
        xhf_ref[:, HID:2 * HID] = hfb
        xhr_ref[:, HID:2 * HID] = hrb
        cf_ref[...] = cf
        cr_ref[...] = cr

    def loop0(j, _):
        step0(2 * j)
        step0(2 * j + 1)
        return 0

    lax.fori_loop(0, T // 2, loop0, 0)

    h1f_ref[...] = zb
    h1r_ref[...] = zb
    cf_ref[...] = zf
    cr_ref[...] = zf

    def step1(i):
        tr = T - 1 - i
        gf = (jnp.dot(ys0_ref[i], wih1f_ref[...], preferred_element_type=f32)
              + jnp.dot(h1f_ref[...], whh1f_ref[...],
                        preferred_element_type=f32)
              + b1f_ref[...])
        gr = (jnp.dot(ys0_ref[tr], wih1r_ref[...], preferred_element_type=f32)
              + jnp.dot(h1r_ref[...], whh1r_ref[...],
                        preferred_element_type=f32)
              + b1r_ref[...])
        hf, cf = update(gf, cf_ref[...])
        hr, cr = update(gr, cr_ref[...])
        h1f_ref[...] = hf.astype(bf16)
        h1r_ref[...] = hr.astype(bf16)
        cf_ref[...] = cf
        cr_ref[...] = cr

    def loop1(j, _):
        step1(2 * j)
        step1(2 * j + 1)
        return 0

    lax.fori_loop(0, T // 2, loop1, 0)

    out_ref[...] = (
        jnp.dot(h1f_ref[...].astype(f32), fcwf_ref[...],
                preferred_element_type=f32)
        + jnp.dot(h1r_ref[...].astype(f32), fcwr_ref[...],
                  preferred_element_type=f32)
        + fcb_ref[...])


def _tc_bilstm(x, w0f, b0f, w0r, b0r,
               wih1f, whh1f, b1f, wih1r, whh1r, b1r, fcwf, fcwr, fcb):
    f32 = jnp.float32
    bf16 = jnp.bfloat16
    full = lambda a: pl.BlockSpec(a.shape, lambda i: (0,) * a.ndim)
    return pl.pallas_call(
        _lstm_body,
        grid=(B // BC,),
        in_specs=[
            pl.BlockSpec((T, BC, EMBP), lambda i: (0, i, 0)),
            full(w0f), full(b0f), full(w0r), full(b0r),
            full(wih1f), full(whh1f), full(b1f),
            full(wih1r), full(whh1r), full(b1r),
            full(fcwf), full(fcwr), full(fcb),
        ],
        out_specs=pl.BlockSpec((BC, OUT), lambda i: (i, 0)),
        out_shape=jax.ShapeDtypeStruct((B, OUT), f32),
        scratch_shapes=[
            pltpu.VMEM((T, BC, 2 * HID), bf16),
            pltpu.VMEM((BC, 2 * HID), bf16),
            pltpu.VMEM((BC, 2 * HID), bf16),
            pltpu.VMEM((BC, HID), f32),
            pltpu.VMEM((BC, HID), f32),
            pltpu.VMEM((BC, HID), bf16),
            pltpu.VMEM((BC, HID), bf16),
        ],
        compiler_params=pltpu.CompilerParams(
            dimension_semantics=("arbitrary",)),
    )(x, w0f, b0f, w0r, b0r, wih1f, whh1f, b1f, wih1r, whh1r, b1r,
      fcwf, fcwr, fcb)


def _gate_scale():
    return jnp.concatenate([
        jnp.full((2 * HID,), 0.5, jnp.float32),
        jnp.ones((HID,), jnp.float32),
        jnp.full((HID,), 0.5, jnp.float32),
    ])


def kernel(text, emb, W_ih_l0, W_hh_l0, b_ih_l0, b_hh_l0, W_ih_l0r, W_hh_l0r,
           b_ih_l0r, b_hh_l0r, W_ih_l1, W_hh_l1, b_ih_l1, b_hh_l1, W_ih_l1r,
           W_hh_l1r, b_ih_l1r, b_hh_l1r, fc_W, fc_b):
    bf16 = jnp.bfloat16
    f32 = jnp.float32
    tt = jnp.transpose(text)
    idx2 = (tt // 2).reshape(1, B * T)
    par = (tt % 2).astype(f32).reshape(T, B, 1)
    emb2 = emb.reshape(emb.shape[0] // 2, 2 * EMB)
    pairs = _sc_gather_pairs(emb2, idx2).reshape(T, B, 2 * EMB)
    x = _sel_cast(pairs, par)

    s = _gate_scale()
    zpad = ((0, EMBP - EMB), (0, 0))
    w0f = jnp.concatenate([jnp.pad(W_ih_l0.T, zpad), W_hh_l0.T], 0) * s
    w0r = jnp.concatenate([jnp.pad(W_ih_l0r.T, zpad), W_hh_l0r.T], 0) * s
    b0f = ((b_ih_l0 + b_hh_l0) * s).reshape(1, 4 * HID)
    b0r = ((b_ih_l0r + b_hh_l0r) * s).reshape(1, 4 * HID)
    b1f = ((b_ih_l1 + b_hh_l1) * s).reshape(1, 4 * HID)
    b1r = ((b_ih_l1r + b_hh_l1r) * s).reshape(1, 4 * HID)
    return _tc_bilstm(
        x,
        w0f.astype(bf16), b0f, w0r.astype(bf16), b0r,
        (W_ih_l1.T * s).astype(bf16), (W_hh_l1.T * s).astype(bf16), b1f,
        (W_ih_l1r.T * s).astype(bf16), (W_hh_l1r.T * s).astype(bf16), b1r,
        fc_W.T[0:HID], fc_W.T[HID:2 * HID], fc_b.reshape(1, OUT),
    )


# speedup vs baseline: 1.7063x; 1.7063x over previous
import jax
import jax.numpy as jnp
from jax import lax
from jax.experimental import pallas as pl
from jax.experimental.pallas import tpu as pltpu
from jax.experimental.pallas import tpu_sc as plsc

T = 50
B = 1024
EMB = 100
EMBP = 128
HID = 128
OUT = 2
BC = 1024
GW = 128


def _sc_gather(emb, idx):
    ni = idx.shape[1]
    mesh = plsc.VectorSubcoreMesh(core_axis_name="core",
                                  subcore_axis_name="subcore")

    @pl.kernel(out_type=jax.ShapeDtypeStruct((ni, EMBP), jnp.float32),
               mesh=mesh)
    def k(emb_hbm, idx_hbm, out_hbm):
        def body(i_vmem, o_vmem):
            pltpu.sync_copy(emb_hbm.at[i_vmem.at[0]], o_vmem)

        pltpu.emit_pipeline(
            body,
            grid=(ni // GW,),
            in_specs=[pl.BlockSpec((1, GW), index_map=lambda i: (0, i))],
            out_specs=[pl.BlockSpec((GW, EMBP),
                                    index_map=lambda i: (i, 0))],
            core_axis_name=("core", "subcore"),
            dimension_semantics=(pltpu.PARALLEL,),
        )(idx_hbm, out_hbm)

    return k(emb, idx)


def _pad_table(emb):
    v = emb.shape[0]
    rows = 2000

    def body(x_ref, o_ref):
        o_ref[:, 0:EMB] = x_ref[...]
        o_ref[:, EMB:EMBP] = jnp.zeros((rows, EMBP - EMB), jnp.float32)

    return pl.pallas_call(
        body,
        grid=(v // rows,),
        in_specs=[pl.BlockSpec((rows, EMB), lambda i: (i, 0))],
        out_specs=pl.BlockSpec((rows, EMBP), lambda i: (i, 0)),
        out_shape=jax.ShapeDtypeStruct((v, EMBP), jnp.float32),
        compiler_params=pltpu.CompilerParams(
            dimension_semantics=("arbitrary",)),
    )(emb)


def _lstm_body(x_ref, w0f_ref, b0f_ref, w0r_ref, b0r_ref,
               wih1f_ref, whh1f_ref, b1f_ref, wih1r_ref, whh1r_ref, b1r_ref,
               fcwf_ref, fcwr_ref, fcb_ref, out_ref,
               ys0_ref, xhf_ref, xhr_ref, cf_ref, cr_ref, h1f_ref, h1r_ref):
    f32 = jnp.float32
    bf16 = jnp.bfloat16

    def update(g, c):
        ti = jnp.tanh(g[:, 0:HID])
        tf = jnp.tanh(g[:, HID:2 * HID])
        tg = jnp.tanh(g[:, 2 * HID:3 * HID])
        to = jnp.tanh(g[:, 3 * HID:4 * HID])
        c = 0.5 * ((tf * c + c) + (ti * tg + tg))
        tc = jnp.tanh(c)
        h = 0.5 * (to * tc + tc)
        return h, c

    zb = jnp.zeros((BC, HID), bf16)
    zf = jnp.zeros((BC, HID), f32)
    xhf_ref[:, HID:2 * HID] = zb
    xhr_ref[:, HID:2 * HID] = zb
    cf_ref[...] = zf
    cr_ref[...] = zf

    def step0(i):
        tr = T - 1 - i
        xhf_ref[:, 0:HID] = x_ref[i].astype(bf16)
        xhr_ref[:, 0:HID] = x_ref[tr].astype(bf16)
        gf = (jnp.dot(xhf_ref[...], w0f_ref[...], preferred_element_type=f32)
              + b0f_ref[...])
        gr = (jnp.dot(xhr_ref[...], w0r_ref[...], preferred_element_type=f32)
              + b0r_ref[...])
        hf, cf = update(gf, cf_ref[...])
        hr, cr = update(gr, cr_ref[...])
        hfb = hf.astype(bf16)
        hrb = hr.astype(bf16)
        ys0_ref[i, :, 0:HID] = hfb
        ys0_ref[tr, :, HID:2 * HID] = hrb
        xhf_ref[:, HID:2 * HID] = hfb
        xhr_ref[:, HID:2 * HID] = hrb
        cf_ref[...] = cf
        cr_ref[...] = cr

    def loop0(j, _):
        step0(2 * j)
        step0(2 * j + 1)
        return 0

    lax.fori_loop(0, T // 2, loop0, 0)

    h1f_ref[...] = zb
    h1r_ref[...] = zb
    cf_ref[...] = zf
    cr_ref[...] = zf

    def step1(i):
        tr = T - 1 - i
        gf = (jnp.dot(ys0_ref[i], wih1f_ref[...], preferred_element_type=f32)
              + jnp.dot(h1f_ref[...], whh1f_ref[...],
                        preferred_element_type=f32)
              + b1f_ref[...])
        gr = (jnp.dot(ys0_ref[tr], wih1r_ref[...], preferred_element_type=f32)
              + jnp.dot(h1r_ref[...], whh1r_ref[...],
                        preferred_element_type=f32)
              + b1r_ref[...])
        hf, cf = update(gf, cf_ref[...])
        hr, cr = update(gr, cr_ref[...])
        h1f_ref[...] = hf.astype(bf16)
        h1r_ref[...] = hr.astype(bf16)
        cf_ref[...] = cf
        cr_ref[...] = cr

    def loop1(j, _):
        step1(2 * j)
        step1(2 * j + 1)
        return 0

    lax.fori_loop(0, T // 2, loop1, 0)

    out_ref[...] = (
        jnp.dot(h1f_ref[...].astype(f32), fcwf_ref[...],
                preferred_element_type=f32)
        + jnp.dot(h1r_ref[...].astype(f32), fcwr_ref[...],
                  preferred_element_type=f32)
        + fcb_ref[...])


def _tc_bilstm(x, w0f, b0f, w0r, b0r,
               wih1f, whh1f, b1f, wih1r, whh1r, b1r, fcwf, fcwr, fcb):
    f32 = jnp.float32
    bf16 = jnp.bfloat16
    full = lambda a: pl.BlockSpec(a.shape, lambda i: (0,) * a.ndim)
    return pl.pallas_call(
        _lstm_body,
        grid=(B // BC,),
        in_specs=[
            pl.BlockSpec((T, BC, EMBP), lambda i: (0, i, 0)),
            full(w0f), full(b0f), full(w0r), full(b0r),
            full(wih1f), full(whh1f), full(b1f),
            full(wih1r), full(whh1r), full(b1r),
            full(fcwf), full(fcwr), full(fcb),
        ],
        out_specs=pl.BlockSpec((BC, OUT), lambda i: (i, 0)),
        out_shape=jax.ShapeDtypeStruct((B, OUT), f32),
        scratch_shapes=[
            pltpu.VMEM((T, BC, 2 * HID), bf16),
            pltpu.VMEM((BC, 2 * HID), bf16),
            pltpu.VMEM((BC, 2 * HID), bf16),
            pltpu.VMEM((BC, HID), f32),
            pltpu.VMEM((BC, HID), f32),
            pltpu.VMEM((BC, HID), bf16),
            pltpu.VMEM((BC, HID), bf16),
        ],
        compiler_params=pltpu.CompilerParams(
            dimension_semantics=("arbitrary",)),
    )(x, w0f, b0f, w0r, b0r, wih1f, whh1f, b1f, wih1r, whh1r, b1r,
      fcwf, fcwr, fcb)


def _gate_scale():
    return jnp.concatenate([
        jnp.full((2 * HID,), 0.5, jnp.float32),
        jnp.ones((HID,), jnp.float32),
        jnp.full((HID,), 0.5, jnp.float32),
    ])


def kernel(text, emb, W_ih_l0, W_hh_l0, b_ih_l0, b_hh_l0, W_ih_l0r, W_hh_l0r,
           b_ih_l0r, b_hh_l0r, W_ih_l1, W_hh_l1, b_ih_l1, b_hh_l1, W_ih_l1r,
           W_hh_l1r, b_ih_l1r, b_hh_l1r, fc_W, fc_b):
    bf16 = jnp.bfloat16
    idx = jnp.transpose(text).reshape(1, B * T)
    emb_p = jnp.pad(emb, ((0, 0), (0, EMBP - EMB)))
    x = _sc_gather(emb_p, idx).reshape(T, B, EMBP)

    s = _gate_scale()
    zpad = ((0, EMBP - EMB), (0, 0))
    w0f = jnp.concatenate([jnp.pad(W_ih_l0.T, zpad), W_hh_l0.T], 0) * s
    w0r = jnp.concatenate([jnp.pad(W_ih_l0r.T, zpad), W_hh_l0r.T], 0) * s
    b0f = ((b_ih_l0 + b_hh_l0) * s).reshape(1, 4 * HID)
    b0r = ((b_ih_l0r + b_hh_l0r) * s).reshape(1, 4 * HID)
    b1f = ((b_ih_l1 + b_hh_l1) * s).reshape(1, 4 * HID)
    b1r = ((b_ih_l1r + b_hh_l1r) * s).reshape(1, 4 * HID)
    return _tc_bilstm(
        x,
        w0f.astype(bf16), b0f, w0r.astype(bf16), b0r,
        (W_ih_l1.T * s).astype(bf16), (W_hh_l1.T * s).astype(bf16), b1f,
        (W_ih_l1r.T * s).astype(bf16), (W_hh_l1r.T * s).astype(bf16), b1r,
        fc_W.T[0:HID], fc_W.T[HID:2 * HID], fc_b.reshape(1, OUT),
    )
